# baseline (speedup 1.0000x reference)
import functools

import jax
import jax.numpy as jnp
from jax import lax
from jax.experimental import pallas as pl
from jax.experimental.pallas import tpu as pltpu
from jax.experimental.pallas import tpu_sc as plsc

N = 100000
E = 1600000
NC = 2
NS = 16
NW = NC * NS
G = 128
EPAD = 1605632
EROWS = EPAD // G
GPT = EROWS // NS
WAVE = 8
NWAVES = GPT // WAVE
DGPT = EROWS // NW
DWAVE = 8
DNWAVES = DGPT // DWAVE
NPAD = 100352
DUMP = N
TROWS = N // NS
K_HOPS = 3
NB = 125
DNB = 625
SET = 4
IB = 8
NBLK = GPT // IB
ZCH = NPAD // NS // G

FLT = jnp.float32


def _zero_accum(zbuf, accum, s, sem):
    def zf(i, _):
        zbuf[i] = jnp.zeros((16,), FLT)
        return 0
    lax.fori_loop(0, G, zf, 0)

    def zc(i, _):
        pltpu.sync_copy(zbuf, accum.at[pl.ds(s * (NPAD // NS) + i * G, G)])
        return 0
    lax.fori_loop(0, ZCH, zc, 0)


def _deg_body(dst2, out, ones_b, didx, stage, zbuf, isem, ssem, wsem, accum):
    c = lax.axis_index("c")
    s = lax.axis_index("s")

    def of(i, _):
        ones_b[i] = jnp.ones((16,), FLT)
        return 0
    lax.fori_loop(0, G, of, 0)
    _zero_accum(zbuf, accum, s, wsem)
    plsc.subcore_barrier()

    base = (c * NS + s) * DGPT

    def blk(b):
        return dst2.at[pl.ds(base + b * DWAVE, DWAVE)]

    def fire_s(p):
        return [
            pltpu.async_copy(
                ones_b, accum.at[didx.at[p].at[k]], ssem, add=True)
            for k in range(DWAVE)
        ]

    pltpu.sync_copy(blk(0), didx.at[0])

    def dit(m, last):
        i1 = pltpu.async_copy(blk(2 * m + 1), didx.at[1], isem)
        sa = fire_s(0)
        i1.wait()
        sb = fire_s(1)
        for d in sa:
            d.wait()
        if not last:
            i0 = pltpu.async_copy(blk(2 * m + 2), didx.at[0], isem)
            i0.wait()
        for d in sb:
            d.wait()

    def dloop(m, _):
        dit(m, False)
        return 0
    lax.fori_loop(0, (DNWAVES - 1) // 2 - 1, dloop, 0)
    dit((DNWAVES - 1) // 2 - 1, True)
    pltpu.sync_copy(blk(DNWAVES - 1), didx.at[0])
    for d in fire_s(0):
        d.wait()
    plsc.subcore_barrier()

    def wb(j, _):
        r = s * TROWS + j * DNB
        pltpu.sync_copy(accum.at[pl.ds(r, DNB)], stage)
        pltpu.sync_copy(stage, out.at[pl.ds(c * N + r, DNB)])
        return 0
    lax.fori_loop(0, TROWS // DNB, wb, 0)


_SC_PARAMS = pltpu.CompilerParams(use_tc_tiling_on_sc=False)


def _deg_call(dst2):
    return pl.kernel(
        _deg_body,
        out_type=jax.ShapeDtypeStruct((NC * N, 16), FLT),
        compiler_params=_SC_PARAMS,
        mesh=plsc.VectorSubcoreMesh(
            core_axis_name="c", subcore_axis_name="s",
            num_cores=NC, num_subcores=NS),
        scratch_types=[
            pltpu.VMEM((G, 16), FLT),
            pltpu.VMEM((2, DWAVE, G), jnp.int32),
            pltpu.VMEM((DNB, 16), FLT),
            pltpu.VMEM((G, 16), FLT),
            pltpu.SemaphoreType.DMA,
            pltpu.SemaphoreType.DMA,
            pltpu.SemaphoreType.DMA,
            pltpu.VMEM_SHARED((NPAD, 16), FLT),
        ],
    )(dst2)


def _spmm_body(C, src2, dst2, table, recip, out,
               ibs, ibd, rows, zbuf, stage, rbuf,
               isem, gsem, ssem, wsem, accum):
    c0 = lax.axis_index("c")
    s = lax.axis_index("s")
    base = s * GPT

    def do_hop(tab, dst_view):
        _zero_accum(zbuf, accum, s, wsem)
        plsc.subcore_barrier()

        def ld_idx(b, p):
            return [
                pltpu.async_copy(
                    src2.at[pl.ds(base + b * IB, IB)], ibs.at[p], isem),
                pltpu.async_copy(
                    dst2.at[pl.ds(base + b * IB, IB)], ibd.at[p], isem),
            ]

        def fire_g(pb, r, rp):
            return pltpu.async_copy(
                tab.at[ibs.at[pb].at[r]], rows.at[rp], gsem)

        def fire_s(pb, r, rp):
            return pltpu.async_copy(
                rows.at[rp], accum.at[ibd.at[pb].at[r]], ssem, add=True)

        def fire_g4(pb, r0, rp):
            return [fire_g(pb, r0 + j, rp * SET + j) for j in range(SET)]

        def fire_s4(pb, r0, rp):
            return [fire_s(pb, r0 + j, rp * SET + j) for j in range(SET)]

        for d in ld_idx(0, 0):
            d.wait()

        def it_body(i, last):
            i1 = ld_idx(2 * i + 1, 1)
            gd0 = fire_g4(0, 0, 0)
            for d in gd0:
                d.wait()
            sd0 = fire_s4(0, 0, 0)
            gd1 = fire_g4(0, SET, 1)
            for d in gd1:
                d.wait()
            sd1 = fire_s4(0, SET, 1)
            for d in sd0:
                d.wait()
            for d in i1:
                d.wait()
            gd2 = fire_g4(1, 0, 0)
            for d in gd2:
                d.wait()
            sd2 = fire_s4(1, 0, 0)
            for d in sd1:
                d.wait()
            gd3 = fire_g4(1, SET, 1)
            for d in gd3:
                d.wait()
            sd3 = fire_s4(1, SET, 1)
            for d in sd2:
                d.wait()
            if not last:
                i0 = ld_idx(2 * i + 2, 0)
                for d in i0:
                    d.wait()
            for d in sd3:
                d.wait()

        def it(i, _):
            it_body(i, False)
            return 0
        lax.fori_loop(0, NBLK // 2 - 1, it, 0)
        it_body(NBLK // 2 - 1, True)
        plsc.subcore_barrier()

        def wb(j, _):
            r = s * TROWS + j * NB
            pltpu.sync_copy(accum.at[pl.ds(r, NB)], stage)
            pltpu.sync_copy(recip.at[pl.ds(r, NB)], rbuf)

            def mul(i, _):
                stage[i] = stage[i] * rbuf[i]
                return 0
            lax.fori_loop(0, NB, mul, 0)
            pltpu.sync_copy(stage, dst_view.at[pl.ds(r, NB)])
            return 0
        lax.fori_loop(0, TROWS // NB, wb, 0)
        plsc.subcore_barrier()

    for cc in range(C // NC):
        ch = cc * NC + c0
        do_hop(table.at[ch], out.at[0, ch])
        for k in range(K_HOPS - 1):
            do_hop(out.at[k, ch], out.at[k + 1, ch])


def _spmm_call(C, src2, dst2, table, recip):
    body = functools.partial(_spmm_body, C)
    return pl.kernel(
        body,
        out_type=jax.ShapeDtypeStruct((K_HOPS, C, N, 16), FLT),
        compiler_params=_SC_PARAMS,
        mesh=plsc.VectorSubcoreMesh(
            core_axis_name="c", subcore_axis_name="s",
            num_cores=NC, num_subcores=NS),
        scratch_types=[
            pltpu.VMEM((2, IB, G), jnp.int32),
            pltpu.VMEM((2, IB, G), jnp.int32),
            pltpu.VMEM((8, G, 16), FLT),
            pltpu.VMEM((G, 16), FLT),
            pltpu.VMEM((NB, 16), FLT),
            pltpu.VMEM((NB, 16), FLT),
            pltpu.SemaphoreType.DMA,
            pltpu.SemaphoreType.DMA,
            pltpu.SemaphoreType.DMA,
            pltpu.SemaphoreType.DMA,
            pltpu.VMEM_SHARED((NPAD, 16), FLT),
        ],
    )(src2, dst2, table, recip)


RB2 = 2000


def _pre_body(cat, noi, degp, h0, recip, sq):
    deg = degp[0] + degp[1]
    pos = deg > 0.0
    dinv = jnp.where(pos, lax.rsqrt(deg), 0.0)
    h0[0] = cat[...] * dinv
    h0[1] = noi[...] * dinv
    recip[...] = jnp.where(pos, 1.0 / deg, 0.0)
    sq[...] = jnp.sqrt(deg)


def _pre_call(cat16, noi16, degp3):
    nb = N // RB2
    return pl.pallas_call(
        _pre_body,
        grid=(nb,),
        in_specs=[
            pl.BlockSpec((RB2, 16), lambda i: (i, 0)),
            pl.BlockSpec((RB2, 16), lambda i: (i, 0)),
            pl.BlockSpec((NC, RB2, 16), lambda i: (0, i, 0)),
        ],
        out_specs=[
            pl.BlockSpec((NC, RB2, 16), lambda i: (0, i, 0)),
            pl.BlockSpec((RB2, 16), lambda i: (i, 0)),
            pl.BlockSpec((RB2, 16), lambda i: (i, 0)),
        ],
        out_shape=[
            jax.ShapeDtypeStruct((NC, N, 16), FLT),
            jax.ShapeDtypeStruct((N, 16), FLT),
            jax.ShapeDtypeStruct((N, 16), FLT),
        ],
    )(cat16, noi16, degp3)


RB = 2000
HID = 64
OUT_DIM = 32


def _mm1_body(cat, noi, hall, sq, w1t, b1, y, h0p):
    sqb = sq[:, 0:1]
    dinv = jnp.where(sqb > 0.0, 1.0 / sqb, 0.0)
    parts = [cat[...], noi[...]]
    for k in range(K_HOPS):
        for cc in range(2):
            parts.append(hall[k, cc] * sqb)
    x = jnp.concatenate(parts, axis=1)
    acc = lax.dot_general(x, w1t[...], (((1,), (0,)), ((), ())),
                          preferred_element_type=FLT) + b1[...]
    yv = jnp.where(acc > 0.0, acc, 0.25 * acc)
    y[...] = yv
    g = yv * dinv
    for cc in range(4):
        h0p[cc] = g[:, cc * 16:(cc + 1) * 16]


def _mm1_call(category, noise, hall, sq16, w1t, b1r):
    nb = N // RB
    return pl.pallas_call(
        _mm1_body,
        grid=(nb,),
        in_specs=[
            pl.BlockSpec((RB, 16), lambda i: (i, 0)),
            pl.BlockSpec((RB, 16), lambda i: (i, 0)),
            pl.BlockSpec((K_HOPS, 2, RB, 16), lambda i: (0, 0, i, 0)),
            pl.BlockSpec((RB, 16), lambda i: (i, 0)),
            pl.BlockSpec((128, HID), lambda i: (0, 0)),
            pl.BlockSpec((1, HID), lambda i: (0, 0)),
        ],
        out_specs=[
            pl.BlockSpec((RB, HID), lambda i: (i, 0)),
            pl.BlockSpec((4, RB, 16), lambda i: (0, i, 0)),
        ],
        out_shape=[
            jax.ShapeDtypeStruct((N, HID), FLT),
            jax.ShapeDtypeStruct((4, N, 16), FLT),
        ],
    )(category, noise, hall, sq16, w1t, b1r)


def _mm2_body(y, gall, sq, w2t, b2, out):
    sqb = sq[:, 0:1]
    parts = [y[...]]
    for k in range(K_HOPS):
        for cc in range(4):
            parts.append(gall[k, cc] * sqb)
    x = jnp.concatenate(parts, axis=1)
    acc = lax.dot_general(x, w2t[...], (((1,), (0,)), ((), ())),
                          preferred_element_type=FLT) + b2[...]
    out[...] = jnp.where(acc > 0.0, acc, 0.25 * acc)


def _mm2_call(y, gall, sq16, w2t, b2r):
    nb = N // RB
    return pl.pallas_call(
        _mm2_body,
        grid=(nb,),
        in_specs=[
            pl.BlockSpec((RB, HID), lambda i: (i, 0)),
            pl.BlockSpec((K_HOPS, 4, RB, 16), lambda i: (0, 0, i, 0)),
            pl.BlockSpec((RB, 16), lambda i: (i, 0)),
            pl.BlockSpec((256, OUT_DIM), lambda i: (0, 0)),
            pl.BlockSpec((1, OUT_DIM), lambda i: (0, 0)),
        ],
        out_specs=pl.BlockSpec((RB, OUT_DIM), lambda i: (i, 0)),
        out_shape=jax.ShapeDtypeStruct((N, OUT_DIM), FLT),
    )(y, gall, sq16, w2t, b2r)


def kernel(category, noise, edge_index, W1, b1, W2, b2):
    src = edge_index[0].astype(jnp.int32)
    dst = edge_index[1].astype(jnp.int32)
    pad = EPAD - E
    src2 = jnp.concatenate(
        [src, jnp.zeros((pad,), jnp.int32)]).reshape(EROWS, G)
    dst2 = jnp.concatenate(
        [dst, jnp.full((pad,), DUMP, jnp.int32)]).reshape(EROWS, G)

    degp = _deg_call(dst2)
    h0, recip16, sq16 = _pre_call(category, noise, degp.reshape(NC, N, 16))
    hall = _spmm_call(2, src2, dst2, h0, recip16)
    y, g0 = _mm1_call(category, noise, hall, sq16,
                      W1.T, b1.reshape(1, HID))
    gall = _spmm_call(4, src2, dst2, g0, recip16)
    out = _mm2_call(y, gall, sq16, W2.T, b2.reshape(1, OUT_DIM))
    return out


# speedup vs baseline: 1.0537x; 1.0537x over previous
import functools

import jax
import jax.numpy as jnp
from jax import lax
from jax.experimental import pallas as pl
from jax.experimental.pallas import tpu as pltpu
from jax.experimental.pallas import tpu_sc as plsc

N = 100000
E = 1600000
NC = 2
NS = 16
NW = NC * NS
G = 128
EPAD = 1605632
EROWS = EPAD // G
GPT = EROWS // NS
WAVE = 8
NWAVES = GPT // WAVE
DGPT = EROWS // NW
DWAVE = 8
DNWAVES = DGPT // DWAVE
NPAD = 100352
DUMP = N
TROWS = N // NS
K_HOPS = 3
NB = 125
DNB = 625
SET = 4
IB = 8
NBLK = GPT // IB
ZCH = NPAD // NS // G

FLT = jnp.float32


def _zero_accum(zbuf, accum, s, sem):
    def zf(i, _):
        zbuf[i] = jnp.zeros((16,), FLT)
        return 0
    lax.fori_loop(0, G, zf, 0)

    def zc(i, _):
        pltpu.sync_copy(zbuf, accum.at[pl.ds(s * (NPAD // NS) + i * G, G)])
        return 0
    lax.fori_loop(0, ZCH, zc, 0)


def _deg_body(dst2, out, ones_b, didx, stage, zbuf, isem, ssem, wsem, accum):
    c = lax.axis_index("c")
    s = lax.axis_index("s")

    def of(i, _):
        ones_b[i] = jnp.ones((16,), FLT)
        return 0
    lax.fori_loop(0, G, of, 0)
    _zero_accum(zbuf, accum, s, wsem)
    plsc.subcore_barrier()

    base = (c * NS + s) * DGPT

    def blk(b):
        return dst2.at[pl.ds(base + b * DWAVE, DWAVE)]

    def fire_s(p):
        return [
            pltpu.async_copy(
                ones_b, accum.at[didx.at[p].at[k]], ssem, add=True)
            for k in range(DWAVE)
        ]

    pltpu.sync_copy(blk(0), didx.at[0])

    def dit(m, last):
        i1 = pltpu.async_copy(blk(2 * m + 1), didx.at[1], isem)
        sa = fire_s(0)
        i1.wait()
        sb = fire_s(1)
        for d in sa:
            d.wait()
        if not last:
            i0 = pltpu.async_copy(blk(2 * m + 2), didx.at[0], isem)
            i0.wait()
        for d in sb:
            d.wait()

    def dloop(m, _):
        dit(m, False)
        return 0
    lax.fori_loop(0, (DNWAVES - 1) // 2 - 1, dloop, 0)
    dit((DNWAVES - 1) // 2 - 1, True)
    pltpu.sync_copy(blk(DNWAVES - 1), didx.at[0])
    for d in fire_s(0):
        d.wait()
    plsc.subcore_barrier()

    def wb(j, _):
        r = s * TROWS + j * DNB
        pltpu.sync_copy(accum.at[pl.ds(r, DNB)], stage)
        pltpu.sync_copy(stage, out.at[pl.ds(c * N + r, DNB)])
        return 0
    lax.fori_loop(0, TROWS // DNB, wb, 0)


_SC_PARAMS = pltpu.CompilerParams(use_tc_tiling_on_sc=False)


def _deg_call(dst2):
    return pl.kernel(
        _deg_body,
        out_type=jax.ShapeDtypeStruct((NC * N, 16), FLT),
        compiler_params=_SC_PARAMS,
        mesh=plsc.VectorSubcoreMesh(
            core_axis_name="c", subcore_axis_name="s",
            num_cores=NC, num_subcores=NS),
        scratch_types=[
            pltpu.VMEM((G, 16), FLT),
            pltpu.VMEM((2, DWAVE, G), jnp.int32),
            pltpu.VMEM((DNB, 16), FLT),
            pltpu.VMEM((G, 16), FLT),
            pltpu.SemaphoreType.DMA,
            pltpu.SemaphoreType.DMA,
            pltpu.SemaphoreType.DMA,
            pltpu.VMEM_SHARED((NPAD, 16), FLT),
        ],
    )(dst2)


def _spmm_body(C, src2, dst2, table, recip, out,
               ibs, ibd, rows, zbuf, stage, rbuf,
               isem, gsem, ssem, wsem, accum):
    c0 = lax.axis_index("c")
    s = lax.axis_index("s")
    base = s * GPT

    def do_hop(tab, dst_view):
        _zero_accum(zbuf, accum, s, wsem)
        plsc.subcore_barrier()

        def ld_idx(b, p):
            return [
                pltpu.async_copy(
                    src2.at[pl.ds(base + b * IB, IB)], ibs.at[p], isem),
                pltpu.async_copy(
                    dst2.at[pl.ds(base + b * IB, IB)], ibd.at[p], isem),
            ]

        def fire_g(pb, r, rp):
            return pltpu.async_copy(
                tab.at[ibs.at[pb].at[r]], rows.at[rp], gsem)

        def fire_s(pb, r, rp):
            return pltpu.async_copy(
                rows.at[rp], accum.at[ibd.at[pb].at[r]], ssem, add=True)

        def fire_g4(pb, r0, rp):
            return [fire_g(pb, r0 + j, rp * SET + j) for j in range(SET)]

        def fire_s4(pb, r0, rp):
            return [fire_s(pb, r0 + j, rp * SET + j) for j in range(SET)]

        for d in ld_idx(0, 0):
            d.wait()

        def it_body(i, last):
            i1 = ld_idx(2 * i + 1, 1)
            gd0 = fire_g4(0, 0, 0)
            for d in gd0:
                d.wait()
            sd0 = fire_s4(0, 0, 0)
            gd1 = fire_g4(0, SET, 1)
            for d in gd1:
                d.wait()
            sd1 = fire_s4(0, SET, 1)
            for d in sd0:
                d.wait()
            for d in i1:
                d.wait()
            gd2 = fire_g4(1, 0, 0)
            for d in gd2:
                d.wait()
            sd2 = fire_s4(1, 0, 0)
            for d in sd1:
                d.wait()
            gd3 = fire_g4(1, SET, 1)
            for d in gd3:
                d.wait()
            sd3 = fire_s4(1, SET, 1)
            for d in sd2:
                d.wait()
            if not last:
                i0 = ld_idx(2 * i + 2, 0)
                for d in i0:
                    d.wait()
            for d in sd3:
                d.wait()

        def it(i, _):
            it_body(i, False)
            return 0
        lax.fori_loop(0, NBLK // 2 - 1, it, 0)
        it_body(NBLK // 2 - 1, True)
        plsc.subcore_barrier()

        def wb(j, _):
            r = s * TROWS + j * NB
            pltpu.sync_copy(accum.at[pl.ds(r, NB)], stage)
            pltpu.sync_copy(recip.at[pl.ds(r, NB)], rbuf)

            def mul(i, _):
                stage[i] = stage[i] * rbuf[i]
                return 0
            lax.fori_loop(0, NB, mul, 0)
            pltpu.sync_copy(stage, dst_view.at[pl.ds(r, NB)])
            return 0
        lax.fori_loop(0, TROWS // NB, wb, 0)
        plsc.subcore_barrier()

    for cc in range(C // NC):
        ch = cc * NC + c0
        do_hop(table.at[ch], out.at[ch])


def _spmm_call(C, src2, dst2, table, recip):
    body = functools.partial(_spmm_body, C)
    return pl.kernel(
        body,
        out_type=jax.ShapeDtypeStruct((C, N, 16), FLT),
        compiler_params=_SC_PARAMS,
        mesh=plsc.VectorSubcoreMesh(
            core_axis_name="c", subcore_axis_name="s",
            num_cores=NC, num_subcores=NS),
        scratch_types=[
            pltpu.VMEM((2, IB, G), jnp.int32),
            pltpu.VMEM((2, IB, G), jnp.int32),
            pltpu.VMEM((8, G, 16), FLT),
            pltpu.VMEM((G, 16), FLT),
            pltpu.VMEM((NB, 16), FLT),
            pltpu.VMEM((NB, 16), FLT),
            pltpu.SemaphoreType.DMA,
            pltpu.SemaphoreType.DMA,
            pltpu.SemaphoreType.DMA,
            pltpu.SemaphoreType.DMA,
            pltpu.VMEM_SHARED((NPAD, 16), FLT),
        ],
    )(src2, dst2, table, recip)


RB2 = 2000


def _pre_body(cat, noi, degp, h0, recip, sq):
    deg = degp[0] + degp[1]
    pos = deg > 0.0
    dinv = jnp.where(pos, lax.rsqrt(deg), 0.0)
    h0[0] = cat[...] * dinv
    h0[1] = noi[...] * dinv
    recip[...] = jnp.where(pos, 1.0 / deg, 0.0)
    sq[...] = jnp.sqrt(deg)


def _pre_call(cat16, noi16, degp3):
    nb = N // RB2
    return pl.pallas_call(
        _pre_body,
        grid=(nb,),
        in_specs=[
            pl.BlockSpec((RB2, 16), lambda i: (i, 0)),
            pl.BlockSpec((RB2, 16), lambda i: (i, 0)),
            pl.BlockSpec((NC, RB2, 16), lambda i: (0, i, 0)),
        ],
        out_specs=[
            pl.BlockSpec((NC, RB2, 16), lambda i: (0, i, 0)),
            pl.BlockSpec((RB2, 16), lambda i: (i, 0)),
            pl.BlockSpec((RB2, 16), lambda i: (i, 0)),
        ],
        out_shape=[
            jax.ShapeDtypeStruct((NC, N, 16), FLT),
            jax.ShapeDtypeStruct((N, 16), FLT),
            jax.ShapeDtypeStruct((N, 16), FLT),
        ],
    )(cat16, noi16, degp3)


RB = 2000
HID = 64
OUT_DIM = 32


def _mm1_body(cat, noi, h1, h2, h3, sq, w1t, b1, y, h0p):
    sqb = sq[:, 0:1]
    dinv = jnp.where(sqb > 0.0, 1.0 / sqb, 0.0)
    parts = [cat[...], noi[...]]
    for h in (h1, h2, h3):
        for cc in range(2):
            parts.append(h[cc] * sqb)
    x = jnp.concatenate(parts, axis=1)
    acc = lax.dot_general(x, w1t[...], (((1,), (0,)), ((), ())),
                          preferred_element_type=FLT) + b1[...]
    yv = jnp.where(acc > 0.0, acc, 0.25 * acc)
    y[...] = yv
    g = yv * dinv
    for cc in range(4):
        h0p[cc] = g[:, cc * 16:(cc + 1) * 16]


def _mm1_call(category, noise, h1, h2, h3, sq16, w1t, b1r):
    nb = N // RB
    tab = lambda: pl.BlockSpec((2, RB, 16), lambda i: (0, i, 0))
    return pl.pallas_call(
        _mm1_body,
        grid=(nb,),
        in_specs=[
            pl.BlockSpec((RB, 16), lambda i: (i, 0)),
            pl.BlockSpec((RB, 16), lambda i: (i, 0)),
            tab(), tab(), tab(),
            pl.BlockSpec((RB, 16), lambda i: (i, 0)),
            pl.BlockSpec((128, HID), lambda i: (0, 0)),
            pl.BlockSpec((1, HID), lambda i: (0, 0)),
        ],
        out_specs=[
            pl.BlockSpec((RB, HID), lambda i: (i, 0)),
            pl.BlockSpec((4, RB, 16), lambda i: (0, i, 0)),
        ],
        out_shape=[
            jax.ShapeDtypeStruct((N, HID), FLT),
            jax.ShapeDtypeStruct((4, N, 16), FLT),
        ],
    )(category, noise, h1, h2, h3, sq16, w1t, b1r)


def _mm2_body(y, g1, g2, g3, sq, w2t, b2, out):
    sqb = sq[:, 0:1]
    parts = [y[...]]
    for g in (g1, g2, g3):
        for cc in range(4):
            parts.append(g[cc] * sqb)
    x = jnp.concatenate(parts, axis=1)
    acc = lax.dot_general(x, w2t[...], (((1,), (0,)), ((), ())),
                          preferred_element_type=FLT) + b2[...]
    out[...] = jnp.where(acc > 0.0, acc, 0.25 * acc)


def _mm2_call(y, g1, g2, g3, sq16, w2t, b2r):
    nb = N // RB
    tab = lambda: pl.BlockSpec((4, RB, 16), lambda i: (0, i, 0))
    return pl.pallas_call(
        _mm2_body,
        grid=(nb,),
        in_specs=[
            pl.BlockSpec((RB, HID), lambda i: (i, 0)),
            tab(), tab(), tab(),
            pl.BlockSpec((RB, 16), lambda i: (i, 0)),
            pl.BlockSpec((256, OUT_DIM), lambda i: (0, 0)),
            pl.BlockSpec((1, OUT_DIM), lambda i: (0, 0)),
        ],
        out_specs=pl.BlockSpec((RB, OUT_DIM), lambda i: (i, 0)),
        out_shape=jax.ShapeDtypeStruct((N, OUT_DIM), FLT),
    )(y, g1, g2, g3, sq16, w2t, b2r)


def kernel(category, noise, edge_index, W1, b1, W2, b2):
    src = edge_index[0].astype(jnp.int32)
    dst = edge_index[1].astype(jnp.int32)
    pad = EPAD - E
    src2 = jnp.concatenate(
        [src, jnp.zeros((pad,), jnp.int32)]).reshape(EROWS, G)
    dst2 = jnp.concatenate(
        [dst, jnp.full((pad,), DUMP, jnp.int32)]).reshape(EROWS, G)

    degp = _deg_call(dst2)
    h0, recip16, sq16 = _pre_call(category, noise, degp.reshape(NC, N, 16))
    h1 = _spmm_call(2, src2, dst2, h0, recip16)
    h2 = _spmm_call(2, src2, dst2, h1, recip16)
    h3 = _spmm_call(2, src2, dst2, h2, recip16)
    y, g0 = _mm1_call(category, noise, h1, h2, h3, sq16,
                      W1.T, b1.reshape(1, HID))
    g1 = _spmm_call(4, src2, dst2, g0, recip16)
    g2 = _spmm_call(4, src2, dst2, g1, recip16)
    g3 = _spmm_call(4, src2, dst2, g2, recip16)
    out = _mm2_call(y, g1, g2, g3, sq16, W2.T, b2.reshape(1, OUT_DIM))
    return out
